# baseline (speedup 1.0000x reference)
import functools

import jax
import jax.numpy as jnp
from jax import lax
from jax.experimental import pallas as pl
from jax.experimental.pallas import tpu as pltpu
from jax.experimental.pallas import tpu_sc as plsc

B, T, D = 4, 4096, 768
N = B * T
NC, NS, L = 2, 16, 16
NW = NC * NS
RPW = N // NW
NG = RPW // L


def _sc_body(x_hbm, m_hbm, emb_hbm, out_hbm,
             mask_v, midx_v, uidx_v, emb16_v, buf_v,
             sem_m, sem_g, sem_s):
    wid = lax.axis_index("s") * NC + lax.axis_index("c")
    base = wid * RPW
    iota = lax.iota(jnp.int32, L)
    zero_v = jnp.zeros((L,), jnp.int32)

    pltpu.sync_copy(m_hbm.at[pl.ds(base, RPW)], mask_v)
    pltpu.sync_copy(emb_hbm, emb16_v)

    def group(g, c):
        mc0, uc0, cmv0, cuv0 = c
        mv = mask_v[pl.ds(g * L, L)]
        mb = jnp.where(mv != 0, jnp.int32(1), jnp.int32(0))
        idx0 = base + g * L
        mc, uc, cmv, cuv = mc0, uc0, cmv0, cuv0
        cnm = zero_v
        cnu = zero_v
        mbase = mc0 & ~(L - 1)
        ubase = uc0 & ~(L - 1)
        for i in range(L):
            ii = idx0 + i
            mi = mb[i]
            pe = (mc - mbase) * mi + (mi - 1) * 100
            cmv = jnp.where(iota == pe, ii, cmv)
            cnm = jnp.where(iota == pe - L, ii, cnm)
            ui = 1 - mi
            qe = (uc - ubase) * ui + (ui - 1) * 100
            cuv = jnp.where(iota == qe, ii, cuv)
            cnu = jnp.where(iota == qe - L, ii, cnu)
            mc = mc + mi
            uc = uc + ui

        cm = (mc >> 4) - (mc0 >> 4)

        @pl.when(cm == 1)
        def _():
            midx_v[pl.ds((mc0 >> 4) * L, L)] = cmv

        cmv = cnm * cm + cmv * (1 - cm)

        cu = (uc >> 4) - (uc0 >> 4)

        @pl.when(cu == 1)
        def _():
            uidx_v[pl.ds((uc0 >> 4) * L, L)] = cuv

        cuv = cnu * cu + cuv * (1 - cu)
        return mc, uc, cmv, cuv

    mc, uc, cmv, cuv = lax.fori_loop(
        0, NG, group, (jnp.int32(0), jnp.int32(0), zero_v, zero_v))

    @pl.when((mc & (L - 1)) != 0)
    def _():
        padded = jnp.where(iota < (mc & (L - 1)), cmv, cmv[0])
        midx_v[pl.ds((mc >> 4) * L, L)] = padded

    @pl.when((uc & (L - 1)) != 0)
    def _():
        padded = jnp.where(iota < (uc & (L - 1)), cuv, cuv[0])
        uidx_v[pl.ds((uc >> 4) * L, L)] = padded

    nmc = (mc + L - 1) >> 4

    def mloop(j, _):
        iv = midx_v[pl.ds(j * L, L)]
        pltpu.async_copy(emb16_v, out_hbm.at[iv], sem_m).wait()
        return 0

    lax.fori_loop(0, nmc, mloop, 0)

    nuc = (uc + L - 1) >> 4

    def uloop(j, _):
        iv = uidx_v[pl.ds(j * L, L)]
        pltpu.async_copy(x_hbm.at[iv], buf_v, sem_g).wait()
        pltpu.async_copy(buf_v, out_hbm.at[iv], sem_s).wait()
        return 0

    lax.fori_loop(0, nuc, uloop, 0)


_sc_masker = functools.partial(
    pl.kernel,
    out_type=jax.ShapeDtypeStruct((N, D), jnp.float32),
    mesh=plsc.VectorSubcoreMesh(
        core_axis_name="c", subcore_axis_name="s",
        num_cores=NC, num_subcores=NS),
    scratch_types=[
        pltpu.VMEM((RPW,), jnp.int32),
        pltpu.VMEM((RPW,), jnp.int32),
        pltpu.VMEM((RPW,), jnp.int32),
        pltpu.VMEM((L, D), jnp.float32),
        pltpu.VMEM((L, D), jnp.float32),
        pltpu.SemaphoreType.DMA,
        pltpu.SemaphoreType.DMA,
        pltpu.SemaphoreType.DMA,
    ],
)(_sc_body)


def kernel(x, mask, mask_emb):
    xf = x.reshape(N, D)
    mi = mask.reshape(N).astype(jnp.int32)
    emb16 = jnp.broadcast_to(mask_emb, (L, D))
    out = _sc_masker(xf, mi, emb16)
    return out.reshape(B, T, D)


# speedup vs baseline: 1.2890x; 1.2890x over previous
import functools

import jax
import jax.numpy as jnp
from jax import lax
from jax.experimental import pallas as pl
from jax.experimental.pallas import tpu as pltpu
from jax.experimental.pallas import tpu_sc as plsc

B, T, D = 4, 4096, 768
N = B * T
NC, NS, L = 2, 16, 16
NW = NC * NS
RPW = N // NW
NG = RPW // L
R = 8
K = 4


def _sc_body(x_hbm, m_hbm, emb_hbm, out_hbm,
             mask_v, uidx_v, emb16_v, buf_v,
             sem_m, sem_g, sem_s):
    wid = lax.axis_index("s") * NC + lax.axis_index("c")
    base = wid * RPW
    iota = lax.iota(jnp.int32, L)
    zero_v = jnp.zeros((L,), jnp.int32)

    cp1 = pltpu.async_copy(m_hbm.at[pl.ds(base, RPW)], mask_v, sem_g)
    cp2 = pltpu.async_copy(emb_hbm, emb16_v, sem_m)
    cp1.wait()
    cp2.wait()

    def buf_slot(c):
        return buf_v.at[pl.ds((c % R) * L, L)]

    def drain_g():
        pltpu.make_async_copy(
            x_hbm.at[pl.ds(0, L)], buf_v.at[pl.ds(0, L)], sem_g).wait()

    def drain_s():
        pltpu.make_async_copy(
            buf_v.at[pl.ds(0, L)], out_hbm.at[pl.ds(0, L)], sem_s).wait()

    def drain_m():
        pltpu.make_async_copy(
            emb16_v, out_hbm.at[pl.ds(0, L)], sem_m).wait()

    def flush_u(c, idx_vec):
        uidx_v[pl.ds(c * L, L)] = idx_vec

        @pl.when(c >= R)
        def _():
            drain_s()

        pltpu.async_copy(x_hbm.at[idx_vec], buf_slot(c), sem_g)

        @pl.when(c >= K)
        def _():
            drain_g()
            iv = uidx_v[pl.ds((c - K) * L, L)]
            pltpu.async_copy(buf_slot(c - K), out_hbm.at[iv], sem_s)

    def group(g, carry):
        mc0, uc0, cmv0, cuv0 = carry
        mv = mask_v[pl.ds(g * L, L)]
        mb = jnp.where(mv != 0, jnp.int32(1), jnp.int32(0))
        idx0 = base + g * L
        mc, uc, cmv, cuv = mc0, uc0, cmv0, cuv0
        cnm = zero_v
        cnu = zero_v
        mbase = mc0 & ~(L - 1)
        ubase = uc0 & ~(L - 1)
        for i in range(L):
            ii = idx0 + i
            mi = mb[i]
            pe = (mc - mbase) * mi + (mi - 1) * 100
            cmv = jnp.where(iota == pe, ii, cmv)
            cnm = jnp.where(iota == pe - L, ii, cnm)
            ui = 1 - mi
            qe = (uc - ubase) * ui + (ui - 1) * 100
            cuv = jnp.where(iota == qe, ii, cuv)
            cnu = jnp.where(iota == qe - L, ii, cnu)
            mc = mc + mi
            uc = uc + ui

        cm = (mc >> 4) - (mc0 >> 4)

        @pl.when(cm == 1)
        def _():
            pltpu.async_copy(emb16_v, out_hbm.at[cmv], sem_m)

        cmv = cnm * cm + cmv * (1 - cm)

        cu = (uc >> 4) - (uc0 >> 4)

        @pl.when(cu == 1)
        def _():
            flush_u(uc0 >> 4, cuv)

        cuv = cnu * cu + cuv * (1 - cu)
        return mc, uc, cmv, cuv

    mc, uc, cmv, cuv = lax.fori_loop(
        0, NG, group, (jnp.int32(0), jnp.int32(0), zero_v, zero_v))

    rem_m = mc & (L - 1)

    @pl.when(rem_m != 0)
    def _():
        padded = jnp.where(iota < rem_m, cmv, cmv[0])
        pltpu.async_copy(emb16_v, out_hbm.at[padded], sem_m)

    rem_u = uc & (L - 1)

    @pl.when(rem_u != 0)
    def _():
        padded = jnp.where(iota < rem_u, cuv, cuv[0])
        flush_u(uc >> 4, padded)

    nu = (uc + L - 1) >> 4
    nm = (mc + L - 1) >> 4

    def tail(j, _):
        drain_g()
        iv = uidx_v[pl.ds(j * L, L)]
        pltpu.async_copy(buf_slot(j), out_hbm.at[iv], sem_s)
        return 0

    lax.fori_loop(jnp.maximum(nu - K, 0), nu, tail, 0)

    def sdrain(j, _):
        drain_s()
        return 0

    lax.fori_loop(0, jnp.minimum(nu, R), sdrain, 0)

    def mdrain(j, _):
        drain_m()
        return 0

    lax.fori_loop(0, nm, mdrain, 0)


_sc_masker = functools.partial(
    pl.kernel,
    out_type=jax.ShapeDtypeStruct((N, D), jnp.float32),
    mesh=plsc.VectorSubcoreMesh(
        core_axis_name="c", subcore_axis_name="s",
        num_cores=NC, num_subcores=NS),
    scratch_types=[
        pltpu.VMEM((RPW,), jnp.int32),
        pltpu.VMEM((RPW,), jnp.int32),
        pltpu.VMEM((L, D), jnp.float32),
        pltpu.VMEM((R * L, D), jnp.float32),
        pltpu.SemaphoreType.DMA,
        pltpu.SemaphoreType.DMA,
        pltpu.SemaphoreType.DMA,
    ],
)(_sc_body)


def kernel(x, mask, mask_emb):
    xf = x.reshape(N, D)
    mi = mask.reshape(N).astype(jnp.int32)
    emb16 = jnp.broadcast_to(mask_emb, (L, D))
    out = _sc_masker(xf, mi, emb16)
    return out.reshape(B, T, D)


# speedup vs baseline: 1.2930x; 1.0031x over previous
import functools

import jax
import jax.numpy as jnp
from jax import lax
from jax.experimental import pallas as pl
from jax.experimental.pallas import tpu as pltpu
from jax.experimental.pallas import tpu_sc as plsc

B, T, D = 4, 4096, 768
N = B * T
NC, NS, L = 2, 16, 16
NW = NC * NS
RPW = N // NW
NG = RPW // L
R = 9
K = 5


def _sc_body(x_hbm, m_hbm, emb_hbm, out_hbm,
             mask_v, uidx_v, emb16_v, buf_v,
             sem_m, sem_g, sem_s):
    wid = lax.axis_index("s") * NC + lax.axis_index("c")
    base = wid * RPW
    iota = lax.iota(jnp.int32, L)
    zero_v = jnp.zeros((L,), jnp.int32)

    cp1 = pltpu.async_copy(m_hbm.at[pl.ds(base, RPW)], mask_v, sem_g)
    cp2 = pltpu.async_copy(emb_hbm, emb16_v, sem_m)
    cp1.wait()
    cp2.wait()

    def buf_slot(c):
        return buf_v.at[pl.ds((c % R) * L, L)]

    def drain_g():
        pltpu.make_async_copy(
            x_hbm.at[pl.ds(0, L)], buf_v.at[pl.ds(0, L)], sem_g).wait()

    def drain_s():
        pltpu.make_async_copy(
            buf_v.at[pl.ds(0, L)], out_hbm.at[pl.ds(0, L)], sem_s).wait()

    def drain_m():
        pltpu.make_async_copy(
            emb16_v, out_hbm.at[pl.ds(0, L)], sem_m).wait()

    def flush_u(c, idx_vec):
        uidx_v[pl.ds(c * L, L)] = idx_vec

        @pl.when(c >= R)
        def _():
            drain_s()

        pltpu.async_copy(x_hbm.at[idx_vec], buf_slot(c), sem_g)

        @pl.when(c >= K)
        def _():
            drain_g()
            iv = uidx_v[pl.ds((c - K) * L, L)]
            pltpu.async_copy(buf_slot(c - K), out_hbm.at[iv], sem_s)

    def group(g, carry):
        mc0, uc0, cmv0, cuv0 = carry
        mv = mask_v[pl.ds(g * L, L)]
        mb = jnp.where(mv != 0, jnp.int32(1), jnp.int32(0))
        idx0 = base + g * L
        mc, uc, cmv, cuv = mc0, uc0, cmv0, cuv0
        cnm = zero_v
        cnu = zero_v
        mbase = mc0 & ~(L - 1)
        ubase = uc0 & ~(L - 1)
        for i in range(L):
            ii = idx0 + i
            mi = mb[i]
            pe = (mc - mbase) * mi + (mi - 1) * 100
            cmv = jnp.where(iota == pe, ii, cmv)
            cnm = jnp.where(iota == pe - L, ii, cnm)
            ui = 1 - mi
            qe = (uc - ubase) * ui + (ui - 1) * 100
            cuv = jnp.where(iota == qe, ii, cuv)
            cnu = jnp.where(iota == qe - L, ii, cnu)
            mc = mc + mi
            uc = uc + ui

        cm = (mc >> 4) - (mc0 >> 4)

        @pl.when(cm == 1)
        def _():
            pltpu.async_copy(emb16_v, out_hbm.at[cmv], sem_m)

        cmv = cnm * cm + cmv * (1 - cm)

        cu = (uc >> 4) - (uc0 >> 4)

        @pl.when(cu == 1)
        def _():
            flush_u(uc0 >> 4, cuv)

        cuv = cnu * cu + cuv * (1 - cu)
        return mc, uc, cmv, cuv

    mc, uc, cmv, cuv = lax.fori_loop(
        0, NG, group, (jnp.int32(0), jnp.int32(0), zero_v, zero_v))

    rem_m = mc & (L - 1)

    @pl.when(rem_m != 0)
    def _():
        padded = jnp.where(iota < rem_m, cmv, cmv[0])
        pltpu.async_copy(emb16_v, out_hbm.at[padded], sem_m)

    rem_u = uc & (L - 1)

    @pl.when(rem_u != 0)
    def _():
        padded = jnp.where(iota < rem_u, cuv, cuv[0])
        flush_u(uc >> 4, padded)

    nu = (uc + L - 1) >> 4
    nm = (mc + L - 1) >> 4

    def tail(j, _):
        drain_g()
        iv = uidx_v[pl.ds(j * L, L)]
        pltpu.async_copy(buf_slot(j), out_hbm.at[iv], sem_s)
        return 0

    lax.fori_loop(jnp.maximum(nu - K, 0), nu, tail, 0)

    def sdrain(j, _):
        drain_s()
        return 0

    lax.fori_loop(0, jnp.minimum(nu, R), sdrain, 0)

    def mdrain(j, _):
        drain_m()
        return 0

    lax.fori_loop(0, nm, mdrain, 0)


_sc_masker = functools.partial(
    pl.kernel,
    out_type=jax.ShapeDtypeStruct((N, D), jnp.float32),
    mesh=plsc.VectorSubcoreMesh(
        core_axis_name="c", subcore_axis_name="s",
        num_cores=NC, num_subcores=NS),
    scratch_types=[
        pltpu.VMEM((RPW,), jnp.int32),
        pltpu.VMEM((RPW,), jnp.int32),
        pltpu.VMEM((L, D), jnp.float32),
        pltpu.VMEM((R * L, D), jnp.float32),
        pltpu.SemaphoreType.DMA,
        pltpu.SemaphoreType.DMA,
        pltpu.SemaphoreType.DMA,
    ],
)(_sc_body)


def kernel(x, mask, mask_emb):
    xf = x.reshape(N, D)
    mi = mask.reshape(N).astype(jnp.int32)
    emb16 = jnp.broadcast_to(mask_emb, (L, D))
    out = _sc_masker(xf, mi, emb16)
    return out.reshape(B, T, D)
